# baseline (speedup 1.0000x reference)
import functools

import jax
import jax.numpy as jnp
from jax import lax
from jax.experimental import pallas as pl
from jax.experimental.pallas import tpu as pltpu
from jax.experimental.pallas import tpu_sc as plsc

_B = 4096
_L = 200
_D = 64
_NW = 32
_ROWS_PER_W = (_B * _L) // _NW
_GROUPS = _ROWS_PER_W // _L
_HALF = _L // 2


def _sc_embed(token_table, idx3, pos_table):
    mesh = plsc.VectorSubcoreMesh(core_axis_name="c", subcore_axis_name="s")

    @functools.partial(
        pl.kernel,
        mesh=mesh,
        compiler_params=pltpu.CompilerParams(use_tc_tiling_on_sc=False),
        out_type=jax.ShapeDtypeStruct((_B * _L, _D), jnp.float32),
        scratch_types=[
            pltpu.VMEM((2 * _GROUPS, _HALF), jnp.int32),
            pltpu.VMEM((_L, _D), jnp.float32),
            pltpu.VMEM((_L, _D), jnp.float32),
            pltpu.SemaphoreType.DMA,
        ],
    )
    def k(table_hbm, idx_hbm, pos_hbm, out_hbm, idx_v, pos_v, rows_v, sem):
        wid = lax.axis_index("s") * 2 + lax.axis_index("c")
        pltpu.sync_copy(idx_hbm.at[wid], idx_v)
        pltpu.sync_copy(pos_hbm, pos_v)

        def group(g, carry):
            c0 = pltpu.async_copy(
                table_hbm.at[idx_v.at[2 * g]], rows_v.at[pl.ds(0, _HALF)], sem)
            c1 = pltpu.async_copy(
                table_hbm.at[idx_v.at[2 * g + 1]], rows_v.at[pl.ds(_HALF, _HALF)], sem)
            c0.wait()
            c1.wait()

            def addrow(r, rcarry):
                for j in range(_D // 16):
                    sl = pl.ds(j * 16, 16)
                    rows_v[r, sl] = rows_v[r, sl] + pos_v[r, sl]
                return rcarry

            lax.fori_loop(0, _L, addrow, 0)
            base = wid * _ROWS_PER_W + g * _L
            pltpu.sync_copy(rows_v, out_hbm.at[pl.ds(base, _L)])
            return carry

        lax.fori_loop(0, _GROUPS, group, 0)

    return k(token_table, idx3, pos_table)


def kernel(seq, token_table, pos_table):
    idx3 = seq.astype(jnp.int32).reshape(_NW, 2 * _GROUPS, _HALF)
    out = _sc_embed(token_table, idx3, pos_table)
    return out.reshape(_B, _L, _D)


# speedup vs baseline: 1.1249x; 1.1249x over previous
import functools

import jax
import jax.numpy as jnp
from jax import lax
from jax.experimental import pallas as pl
from jax.experimental.pallas import tpu as pltpu
from jax.experimental.pallas import tpu_sc as plsc

_B = 4096
_L = 200
_D = 64
_NW = 32
_ROWS_PER_W = (_B * _L) // _NW
_GROUPS = _ROWS_PER_W // _L
_HALF = _L // 2
_NBUF = 4


def _sc_embed(token_table, idx3, pos_table):
    mesh = plsc.VectorSubcoreMesh(core_axis_name="c", subcore_axis_name="s")

    @functools.partial(
        pl.kernel,
        mesh=mesh,
        compiler_params=pltpu.CompilerParams(use_tc_tiling_on_sc=False),
        out_type=jax.ShapeDtypeStruct((_B * _L, _D), jnp.float32),
        scratch_types=[
            pltpu.VMEM((2 * _GROUPS, _HALF), jnp.int32),
            pltpu.VMEM((_L, _D), jnp.float32),
            pltpu.VMEM((_NBUF, _L, _D), jnp.float32),
        ]
        + [pltpu.SemaphoreType.DMA] * (2 * _NBUF),
    )
    def k(table_hbm, idx_hbm, pos_hbm, out_hbm, idx_v, pos_v, rows_v, *sems):
        gsems = sems[:_NBUF]
        wsems = sems[_NBUF:]
        wid = lax.axis_index("s") * 2 + lax.axis_index("c")
        base_w = wid * _ROWS_PER_W
        pltpu.sync_copy(idx_hbm.at[wid], idx_v)
        pltpu.sync_copy(pos_hbm, pos_v)

        def start_gather(g, b):
            pltpu.async_copy(
                table_hbm.at[idx_v.at[2 * g]],
                rows_v.at[b, pl.ds(0, _HALF)], gsems[b])
            pltpu.async_copy(
                table_hbm.at[idx_v.at[2 * g + 1]],
                rows_v.at[b, pl.ds(_HALF, _HALF)], gsems[b])

        def wait_gather(b):
            for h in range(2):
                pltpu.make_async_copy(
                    table_hbm.at[idx_v.at[h]],
                    rows_v.at[b, pl.ds(h * _HALF, _HALF)], gsems[b]).wait()

        def start_write(g, b):
            pltpu.async_copy(
                rows_v.at[b], out_hbm.at[pl.ds(base_w + g * _L, _L)], wsems[b])

        def wait_write(b):
            pltpu.make_async_copy(
                rows_v.at[b], out_hbm.at[pl.ds(0, _L)], wsems[b]).wait()

        def add_pos(b):
            @plsc.parallel_loop(0, _L, unroll=4)
            def _addrow(r):
                for j in range(_D // 16):
                    sl = pl.ds(j * 16, 16)
                    rows_v[b, r, sl] = rows_v[b, r, sl] + pos_v[r, sl]

        for b in range(_NBUF - 1):
            start_gather(b, b)

        def outer(oi, carry):
            for b in range(_NBUF):
                g = oi * _NBUF + b
                b3 = (b + _NBUF - 1) % _NBUF

                @pl.when(g >= 1)
                def _():
                    wait_write(b3)

                @pl.when(g + _NBUF - 1 < _GROUPS)
                def _():
                    start_gather(g + _NBUF - 1, b3)

                wait_gather(b)
                add_pos(b)
                start_write(g, b)
            return carry

        lax.fori_loop(0, _GROUPS // _NBUF, outer, 0)
        wait_write((_GROUPS - 1) % _NBUF)

    return k(token_table, idx3, pos_table)


def kernel(seq, token_table, pos_table):
    idx3 = seq.astype(jnp.int32).reshape(_NW, 2 * _GROUPS, _HALF)
    out = _sc_embed(token_table, idx3, pos_table)
    return out.reshape(_B, _L, _D)


# speedup vs baseline: 1.1260x; 1.0010x over previous
import functools

import jax
import jax.numpy as jnp
from jax import lax
from jax.experimental import pallas as pl
from jax.experimental.pallas import tpu as pltpu
from jax.experimental.pallas import tpu_sc as plsc

_B = 4096
_L = 200
_D = 64
_NW = 32
_BROWS = _B // _NW
_CHUNK = 40
_NCH = _L // _CHUNK
_NBUF = 4


def _sc_embed(seq, token_table, pos_table):
    mesh = plsc.VectorSubcoreMesh(core_axis_name="c", subcore_axis_name="s")

    @functools.partial(
        pl.kernel,
        mesh=mesh,
        compiler_params=pltpu.CompilerParams(use_tc_tiling_on_sc=False),
        out_type=jax.ShapeDtypeStruct((_B, _L, _D), jnp.float32),
        scratch_types=[
            pltpu.VMEM((_BROWS, _L), jnp.int32),
            pltpu.VMEM((_L, _D), jnp.float32),
            pltpu.VMEM((_NBUF, _L, _D), jnp.float32),
        ]
        + [pltpu.SemaphoreType.DMA] * (2 * _NBUF),
    )
    def k(table_hbm, idx_hbm, pos_hbm, out_hbm, idx_v, pos_v, rows_v, *sems):
        gsems = sems[:_NBUF]
        wsems = sems[_NBUF:]
        wid = lax.axis_index("s") * 2 + lax.axis_index("c")
        base_b = wid * _BROWS
        pltpu.sync_copy(idx_hbm.at[pl.ds(base_b, _BROWS)], idx_v)
        pltpu.sync_copy(pos_hbm, pos_v)

        def start_gather(g, b):
            for c in range(_NCH):
                pltpu.async_copy(
                    table_hbm.at[idx_v.at[g, pl.ds(c * _CHUNK, _CHUNK)]],
                    rows_v.at[b, pl.ds(c * _CHUNK, _CHUNK)], gsems[b])

        def wait_gather(b):
            for c in range(_NCH):
                pltpu.make_async_copy(
                    table_hbm.at[idx_v.at[0, pl.ds(0, _CHUNK)]],
                    rows_v.at[b, pl.ds(c * _CHUNK, _CHUNK)], gsems[b]).wait()

        def start_write(g, b):
            pltpu.async_copy(rows_v.at[b], out_hbm.at[base_b + g], wsems[b])

        def wait_write(b):
            pltpu.make_async_copy(rows_v.at[b], out_hbm.at[0], wsems[b]).wait()

        def add_pos(b):
            @plsc.parallel_loop(0, _L, unroll=4)
            def _addrow(r):
                for j in range(_D // 16):
                    sl = pl.ds(j * 16, 16)
                    rows_v[b, r, sl] = rows_v[b, r, sl] + pos_v[r, sl]

        for b in range(_NBUF - 1):
            start_gather(b, b)

        def outer(oi, carry):
            for b in range(_NBUF):
                g = oi * _NBUF + b
                b3 = (b + _NBUF - 1) % _NBUF

                @pl.when(g >= 1)
                def _():
                    wait_write(b3)

                @pl.when(g + _NBUF - 1 < _BROWS)
                def _():
                    start_gather(g + _NBUF - 1, b3)

                wait_gather(b)
                add_pos(b)
                start_write(g, b)
            return carry

        lax.fori_loop(0, _BROWS // _NBUF, outer, 0)
        wait_write((_BROWS - 1) % _NBUF)

    return k(token_table, seq, pos_table)


def kernel(seq, token_table, pos_table):
    return _sc_embed(seq.astype(jnp.int32), token_table, pos_table)
